# Initial kernel scaffold; baseline (speedup 1.0000x reference)
import jax
import jax.numpy as jnp
from jax.experimental import pallas as pl


def kernel(x, graph_batch, W1, b1, W2, b2):
    raise NotImplementedError("write your pallas kernel here")


# speedup vs baseline: 2.0255x; 2.0255x over previous
import jax
import jax.numpy as jnp
from jax.experimental import pallas as pl


def _gcn_kernel(x_ref, adj_ref, w1_ref, b1_ref, w2_ref, b2_ref, out_ref):
    adj = adj_ref[0]
    t = jnp.dot(x_ref[0], w1_ref[...],
                preferred_element_type=jnp.float32)
    h = jnp.dot(adj, t, preferred_element_type=jnp.float32) + b1_ref[...]
    g = jnp.where(h >= 0.0, h, 0.01 * h)
    n = adj.shape[0]
    r = jnp.sum(adj, axis=0) * (1.0 / n)
    v = jnp.sum(g * r[:, None], axis=0)
    out_ref[0] = (jnp.dot(v[None, :], w2_ref[...],
                          preferred_element_type=jnp.float32)
                  + b2_ref[...])


def kernel(x, graph_batch, W1, b1, W2, b2):
    B, N, d_in = x.shape
    d_hid = W1.shape[1]
    d_out = W2.shape[1]
    b1r = b1.reshape(1, d_hid)
    b2r = b2.reshape(1, d_out)
    return pl.pallas_call(
        _gcn_kernel,
        grid=(B,),
        in_specs=[
            pl.BlockSpec((1, N, d_in), lambda b: (b, 0, 0)),
            pl.BlockSpec((1, N, N), lambda b: (b, 0, 0)),
            pl.BlockSpec((d_in, d_hid), lambda b: (0, 0)),
            pl.BlockSpec((1, d_hid), lambda b: (0, 0)),
            pl.BlockSpec((d_hid, d_out), lambda b: (0, 0)),
            pl.BlockSpec((1, d_out), lambda b: (0, 0)),
        ],
        out_specs=pl.BlockSpec((1, 1, d_out), lambda b: (b, 0, 0)),
        out_shape=jax.ShapeDtypeStruct((B, 1, d_out), jnp.float32),
    )(x, graph_batch, W1, b1r, W2, b2r).reshape(B, d_out)
